# Initial kernel scaffold; baseline (speedup 1.0000x reference)
import jax
import jax.numpy as jnp
from jax.experimental import pallas as pl


def kernel(input_ids, tables):
    raise NotImplementedError("write your pallas kernel here")


# speedup vs baseline: 7.5788x; 7.5788x over previous
import functools
import math

import jax
import jax.numpy as jnp
from jax import lax
from jax.experimental import pallas as pl
from jax.experimental.pallas import tpu as pltpu
from jax.experimental.pallas import tpu_sc as plsc

_NUM_ORDERS = 3
_FEATURES = 128
_NUM_EMB = 100000
_NUM_TABLES = 8
_SHARD = _FEATURES // _NUM_TABLES
_MULT = 1000003
_PRIMES = (2, 3, 5, 7, 11, 13, 17, 19)

_B = 1024
_T = 200
_L = 16
_NC = 2
_NS = 16
_NW = _NC * _NS
_ROWS_PER_WORKER = _B // _NW
_GROUPS = 13
_TPAD = _GROUPS * _L
_RPC = _TPAD * _NUM_TABLES
_ROWS_OUT = _T * _NUM_TABLES
_SUB = 128
_NSUB = _RPC // _SUB


def _sc_body(ids_hbm, table_hbm, out_hbm,
             ids_v, idx0, idx1, idx2, buf0, buf1, buf2, sem):
    wid = lax.axis_index("c") * _NS + lax.axis_index("s")
    iota = lax.iota(jnp.int32, _L)

    idx_refs = (idx0, idx1, idx2)
    bufs = (buf0, buf1, buf2)

    def row_body(k, carry):
        r = wid * _ROWS_PER_WORKER + k
        pltpu.sync_copy(ids_hbm.at[r], ids_v)

        def grp(g, c2):
            t0 = pl.multiple_of(g * _L, _L)
            a = ids_v[pl.ds(t0, _L)]
            b = ids_v[pl.ds(t0 + 1, _L)]
            c = ids_v[pl.ds(t0 + 2, _L)]
            fp2 = a * _MULT + b
            fp3 = fp2 * _MULT + c
            gvec = jnp.full((_L,), 0, jnp.int32) + g
            for oi, fp in enumerate((a, fp2, fp3)):
                fpp = fp + 1
                for ti in range(_NUM_TABLES):
                    v = fpp * _PRIMES[ti]
                    h = lax.rem(v, _NUM_EMB)
                    h = jnp.where(h < 0, h + _NUM_EMB, h)
                    h = h + ((oi * _NUM_TABLES + ti) * _NUM_EMB)
                    plsc.store_scatter(
                        idx_refs[oi], [gvec, iota * _NUM_TABLES + ti], h)
            return c2

        lax.fori_loop(0, _GROUPS, grp, 0)

        copies = []
        for oi in range(_NUM_ORDERS):
            for j in range(_NSUB):
                copies.append(pltpu.async_copy(
                    table_hbm.at[idx_refs[oi].at[j]],
                    bufs[oi].at[pl.ds(j * _SUB, _SUB)], sem))
        for cp in copies:
            cp.wait()

        def acc(i, c3):
            buf0[i, :] = buf0[i, :] + buf1[i, :] + buf2[i, :]
            return c3

        lax.fori_loop(0, _ROWS_OUT, acc, 0)

        pltpu.sync_copy(buf0.at[pl.ds(0, _ROWS_OUT)],
                        out_hbm.at[pl.ds(r * _ROWS_OUT, _ROWS_OUT)])
        return carry

    lax.fori_loop(0, _ROWS_PER_WORKER, row_body, 0)


@jax.jit
def _ngram_embed_sc(input_ids, table_flat):
    mesh = plsc.VectorSubcoreMesh(core_axis_name="c", subcore_axis_name="s")
    fn = functools.partial(
        pl.kernel,
        out_type=jax.ShapeDtypeStruct((_B * _ROWS_OUT, _SHARD), jnp.float32),
        mesh=mesh,
        compiler_params=pltpu.CompilerParams(
            needs_layout_passes=False, use_tc_tiling_on_sc=False),
        scratch_types=[
            pltpu.VMEM((256,), jnp.int32),
            pltpu.VMEM((_NSUB, _SUB), jnp.int32),
            pltpu.VMEM((_NSUB, _SUB), jnp.int32),
            pltpu.VMEM((_NSUB, _SUB), jnp.int32),
            pltpu.VMEM((_RPC, _SHARD), jnp.float32),
            pltpu.VMEM((_RPC, _SHARD), jnp.float32),
            pltpu.VMEM((_RPC, _SHARD), jnp.float32),
            pltpu.SemaphoreType.DMA,
        ],
    )(_sc_body)
    return fn(input_ids, table_flat)


def kernel(input_ids, tables):
    table_flat = tables.reshape(_NUM_ORDERS * _NUM_TABLES * _NUM_EMB, _SHARD)
    ids_pad = jnp.zeros((_B, 256), jnp.int32).at[:, :_T].set(
        input_ids.astype(jnp.int32))
    out = _ngram_embed_sc(ids_pad, table_flat)
    return out.reshape(_B, _T, _FEATURES)
